# Initial kernel scaffold; baseline (speedup 1.0000x reference)
import jax
import jax.numpy as jnp
from jax.experimental import pallas as pl


def kernel(input_ids, table, rotation_matrix, group_scale, group_bias):
    raise NotImplementedError("write your pallas kernel here")


# speedup vs baseline: 22.4998x; 22.4998x over previous
import functools

import jax
import jax.numpy as jnp
from jax import lax
from jax.experimental import pallas as pl
from jax.experimental.pallas import tpu as pltpu
from jax.experimental.pallas import tpu_sc as plsc

H = 8
GROUP = 8


def _transform_body(x_ref, m_ref, s_ref, b_ref, o_ref):
    x = x_ref[...]
    y = jnp.dot(x, m_ref[...], preferred_element_type=jnp.float32)
    o_ref[...] = y * s_ref[0, 0] + b_ref[...]


def _transform_table(table, rotation_matrix, group_scale, group_bias):
    V = table.shape[0]
    rows = V // GROUP
    x = table.reshape(rows, GROUP * H)
    eye = jnp.eye(GROUP, dtype=table.dtype)
    big_r = jnp.kron(eye, rotation_matrix)
    bias_row = jnp.tile(group_bias, GROUP).reshape(1, GROUP * H)
    scale = group_scale.reshape(1, 1)
    grid = 25
    blk = rows // grid
    out = pl.pallas_call(
        _transform_body,
        grid=(grid,),
        in_specs=[
            pl.BlockSpec((blk, GROUP * H), lambda i: (i, 0)),
            pl.BlockSpec((GROUP * H, GROUP * H), lambda i: (0, 0)),
            pl.BlockSpec((1, 1), lambda i: (0, 0)),
            pl.BlockSpec((1, GROUP * H), lambda i: (0, 0)),
        ],
        out_specs=pl.BlockSpec((blk, GROUP * H), lambda i: (i, 0)),
        out_shape=jax.ShapeDtypeStruct((rows, GROUP * H), jnp.float32),
    )(x, big_r, scale, bias_row)
    return out.reshape(V, H)


def _make_gather(N, V):
    info = plsc.get_sparse_core_info()
    NC, NS = info.num_cores, info.num_subcores
    NW = NC * NS
    per_w = N // NW
    C = 4096
    n_chunks = per_w // C
    mesh = plsc.VectorSubcoreMesh(core_axis_name="c", subcore_axis_name="s")

    @functools.partial(
        pl.kernel,
        out_type=jax.ShapeDtypeStruct((N, H), jnp.float32),
        mesh=mesh,
        compiler_params=pltpu.CompilerParams(use_tc_tiling_on_sc=False),
        scratch_types=[
            pltpu.VMEM((C,), jnp.int32),
            pltpu.VMEM((C, H), jnp.float32),
            pltpu.SemaphoreType.DMA,
        ],
    )
    def gather_kernel(ids_hbm, tbl_hbm, out_hbm, idx_v, rows_v, sem):
        wid = lax.axis_index("s") * NC + lax.axis_index("c")
        base = wid * per_w

        def body(j, carry):
            off = base + j * C
            pltpu.sync_copy(ids_hbm.at[pl.ds(off, C)], idx_v)
            pltpu.async_copy(tbl_hbm.at[idx_v], rows_v, sem).wait()
            pltpu.sync_copy(rows_v, out_hbm.at[pl.ds(off, C)])
            return carry

        lax.fori_loop(0, n_chunks, body, 0)

    return gather_kernel


def kernel(input_ids, table, rotation_matrix, group_scale, group_bias):
    B, S = input_ids.shape
    V = table.shape[0]
    N = B * S
    t_prime = _transform_table(table, rotation_matrix, group_scale, group_bias)
    ids = input_ids.reshape(N).astype(jnp.int32)
    out = _make_gather(N, V)(ids, t_prime)
    return out.reshape(B, S, H)
